# baseline (speedup 1.0000x reference)
import functools

import jax
import jax.numpy as jnp
from jax import lax
from jax.experimental import pallas as pl
from jax.experimental.pallas import tpu as pltpu
from jax.experimental.pallas import tpu_sc as plsc

N = 10000
E = 320000
D = 128
ED = 16
H = 128
OUT = 128
B = 256

NC = 2
NS = 16
NW = NC * NS
CH = 40
NBUF = 3
GRP = 24
NG = 11
NCHUNK = GRP * NG
EPW = NCHUNK * CH
EP = NW * EPW
NP_ = 10112
DUMP = N
RPS = NP_ // NS


def _ln(t, g, b, eps=1e-5):
    m = jnp.mean(t, axis=-1, keepdims=True)
    v = jnp.mean((t - m) ** 2, axis=-1, keepdims=True)
    return (t - m) / jnp.sqrt(v + eps) * g + b


_BE = 2048


def _edge_embed_body(ea_ref, We0_ref, bE0_ref, We1_ref, bE1_ref,
                     ee0_ref, ee1_ref):
    ea = ea_ref[...]
    ee0_ref[...] = jnp.dot(ea, We0_ref[...],
                           preferred_element_type=jnp.float32) + bE0_ref[...]
    ee1_ref[...] = jnp.dot(ea, We1_ref[...],
                           preferred_element_type=jnp.float32) + bE1_ref[...]


def _edge_embed(edge_attr, We0, bE0, We1, bE1):
    grid = (EP // _BE,)
    return pl.pallas_call(
        _edge_embed_body,
        grid=grid,
        in_specs=[
            pl.BlockSpec((_BE, ED), lambda i: (i, 0)),
            pl.BlockSpec((ED, D), lambda i: (0, 0)),
            pl.BlockSpec((1, D), lambda i: (0, 0)),
            pl.BlockSpec((ED, D), lambda i: (0, 0)),
            pl.BlockSpec((1, D), lambda i: (0, 0)),
        ],
        out_specs=[
            pl.BlockSpec((_BE, D), lambda i: (i, 0)),
            pl.BlockSpec((_BE, D), lambda i: (i, 0)),
        ],
        out_shape=[
            jax.ShapeDtypeStruct((EP, D), jnp.float32),
            jax.ShapeDtypeStruct((EP, D), jnp.float32),
        ],
    )(edge_attr, We0, bE0.reshape(1, D), We1, bE1.reshape(1, D))


def _msgpass_body(x_hbm, ee_hbm, src_hbm, dst_hbm, out_hbm,
                  src_g, dst_g, rows, eev, shared,
                  se0, se1, se2, sg0, sg1, sg2, ss0, ss1, ss2):
    cid = lax.axis_index("c")
    sid = lax.axis_index("s")
    wid = cid * NS + sid
    esems = (se0, se1, se2)
    gsems = (sg0, sg1, sg2)
    ssems = (ss0, ss1, ss2)

    def zrow(r, _):
        for j in range(D // 16):
            rows[0, r, pl.ds(j * 16, 16)] = jnp.zeros((16,), jnp.float32)
        return 0
    lax.fori_loop(0, CH, zrow, 0)
    for k in range(RPS // CH):
        pltpu.sync_copy(rows.at[0], shared.at[pl.ds(sid * RPS + k * CH, CH)])
    rem = RPS % CH
    if rem:
        pltpu.sync_copy(rows.at[0, pl.ds(0, rem)],
                        shared.at[pl.ds(sid * RPS + (RPS // CH) * CH, rem)])

    plsc.subcore_barrier()

    NI = GRP // NBUF

    def group(g, _):
        pltpu.sync_copy(src_hbm.at[wid, pl.ds(g * GRP, GRP)], src_g)
        pltpu.sync_copy(dst_hbm.at[wid, pl.ds(g * GRP, GRP)], dst_g)

        def issue(cg, b):
            pltpu.async_copy(
                ee_hbm.at[pl.ds(wid * EPW + (g * GRP + cg) * CH, CH)],
                eev.at[b], esems[b])
            pltpu.async_copy(x_hbm.at[src_g.at[cg]], rows.at[b], gsems[b])

        for b in range(NBUF):
            issue(b, b)

        def inner(gg, _):
            for b in range(NBUF):
                cg = gg * NBUF + b
                pltpu.make_async_copy(ee_hbm.at[pl.ds(0, CH)], eev.at[b],
                                      esems[b]).wait()
                pltpu.make_async_copy(x_hbm.at[pl.ds(0, CH)], rows.at[b],
                                      gsems[b]).wait()
                def row(r, _):
                    for jj in range(D // 16):
                        sl = pl.ds(jj * 16, 16)
                        rows[b, r, sl] = jnp.maximum(
                            rows[b, r, sl] + eev[b, r, sl], 0.0)
                    return 0
                lax.fori_loop(0, CH, row, 0)
                bp = (b + NBUF - 1) % NBUF

                def recycle():
                    pltpu.make_async_copy(x_hbm.at[pl.ds(0, CH)],
                                          rows.at[bp], ssems[bp]).wait()

                    def refill():
                        issue(cg + NBUF - 1, bp)
                    if b == 0:
                        refill()
                    else:
                        pl.when(gg < NI - 1)(refill)
                if b == 0:
                    pl.when(gg > 0)(recycle)
                else:
                    recycle()
                pltpu.async_copy(rows.at[b], shared.at[dst_g.at[cg]],
                                 ssems[b], add=True)
            return 0

        lax.fori_loop(0, NI, inner, 0)

        pltpu.make_async_copy(x_hbm.at[pl.ds(0, CH)], rows.at[NBUF - 1],
                              ssems[NBUF - 1]).wait()
        return 0

    lax.fori_loop(0, NG, group, 0)

    plsc.subcore_barrier()

    pltpu.sync_copy(shared.at[pl.ds(sid * RPS, RPS)],
                    out_hbm.at[cid, pl.ds(sid * RPS, RPS)])


def _msgpass(x, ee, src3, dst3):
    mesh = plsc.VectorSubcoreMesh(core_axis_name="c", subcore_axis_name="s",
                                  num_cores=NC, num_subcores=NS)
    f = pl.kernel(
        _msgpass_body,
        out_type=jax.ShapeDtypeStruct((NC, NP_, D), jnp.float32),
        mesh=mesh,
        scratch_types=[
            pltpu.VMEM((GRP, CH), jnp.int32),
            pltpu.VMEM((GRP, CH), jnp.int32),
            pltpu.VMEM((NBUF, CH, D), jnp.float32),
            pltpu.VMEM((NBUF, CH, D), jnp.float32),
            pltpu.VMEM_SHARED((NP_, D), jnp.float32),
        ] + [pltpu.SemaphoreType.DMA] * 9,
    )
    return f(x, ee, src3, dst3)


_BN = 2000


def _node_mlp_body(x_ref, p_ref, W1_ref, b1_ref, g1_ref, be1_ref,
                   W2_ref, b2_ref, g2_ref, be2_ref, o_ref):
    h = x_ref[...] + p_ref[0] + p_ref[1]
    t = jnp.dot(h, W1_ref[...], preferred_element_type=jnp.float32) + b1_ref[...]
    t = jnp.maximum(_ln(t, g1_ref[...], be1_ref[...]), 0.0)
    u = jnp.dot(t, W2_ref[...], preferred_element_type=jnp.float32) + b2_ref[...]
    o_ref[...] = jnp.maximum(_ln(u, g2_ref[...], be2_ref[...]), 0.0)


def _node_mlp(x, p, W1, b1, g1, be1, W2, b2, g2, be2):
    grid = (N // _BN,)
    row = lambda v: v.reshape(1, D)
    return pl.pallas_call(
        _node_mlp_body,
        grid=grid,
        in_specs=[
            pl.BlockSpec((_BN, D), lambda i: (i, 0)),
            pl.BlockSpec((NC, _BN, D), lambda i: (0, i, 0)),
            pl.BlockSpec((D, D), lambda i: (0, 0)),
            pl.BlockSpec((1, D), lambda i: (0, 0)),
            pl.BlockSpec((1, D), lambda i: (0, 0)),
            pl.BlockSpec((1, D), lambda i: (0, 0)),
            pl.BlockSpec((D, D), lambda i: (0, 0)),
            pl.BlockSpec((1, D), lambda i: (0, 0)),
            pl.BlockSpec((1, D), lambda i: (0, 0)),
            pl.BlockSpec((1, D), lambda i: (0, 0)),
        ],
        out_specs=pl.BlockSpec((_BN, D), lambda i: (i, 0)),
        out_shape=jax.ShapeDtypeStruct((N, D), jnp.float32),
    )(x, p, W1, row(b1), row(g1), row(be1), W2, row(b2), row(g2), row(be2))


def _final_body(x_ref, p_ref, batch_ref, W1_ref, b1_ref, g1_ref, be1_ref,
                W2_ref, b2_ref, g2_ref, be2_ref, Wp_ref, bp_ref,
                o_ref, pooled_acc, counts_acc):
    i = pl.program_id(0)
    h = x_ref[...] + p_ref[0] + p_ref[1]
    t = jnp.dot(h, W1_ref[...], preferred_element_type=jnp.float32) + b1_ref[...]
    t = jnp.maximum(_ln(t, g1_ref[...], be1_ref[...]), 0.0)
    u = jnp.dot(t, W2_ref[...], preferred_element_type=jnp.float32) + b2_ref[...]
    x2 = jnp.maximum(_ln(u, g2_ref[...], be2_ref[...]), 0.0)

    bvec = batch_ref[0, 0, :]
    mask = (bvec[:, None] ==
            lax.broadcasted_iota(jnp.int32, (_BN, B), 1)).astype(jnp.float32)
    pm = lax.dot_general(mask, x2, (((0,), (0,)), ((), ())),
                         preferred_element_type=jnp.float32)
    cm = jnp.sum(mask, axis=0)[None, :]

    @pl.when(i == 0)
    def _():
        pooled_acc[...] = jnp.zeros_like(pooled_acc)
        counts_acc[...] = jnp.zeros_like(counts_acc)

    pooled_acc[...] += pm
    counts_acc[...] += cm

    @pl.when(i == pl.num_programs(0) - 1)
    def _():
        cnt = jnp.maximum(counts_acc[0, :], 1.0)
        proj = jnp.dot(pooled_acc[...], Wp_ref[...],
                       preferred_element_type=jnp.float32)
        o_ref[...] = proj / cnt[:, None] + bp_ref[...]


def _final(x, p, batch, W1, b1, g1, be1, W2, b2, g2, be2, Wp, bp):
    grid = (N // _BN,)
    row = lambda v: v.reshape(1, D)
    return pl.pallas_call(
        _final_body,
        grid=grid,
        in_specs=[
            pl.BlockSpec((_BN, D), lambda i: (i, 0)),
            pl.BlockSpec((NC, _BN, D), lambda i: (0, i, 0)),
            pl.BlockSpec((1, 1, _BN), lambda i: (i, 0, 0)),
            pl.BlockSpec((D, D), lambda i: (0, 0)),
            pl.BlockSpec((1, D), lambda i: (0, 0)),
            pl.BlockSpec((1, D), lambda i: (0, 0)),
            pl.BlockSpec((1, D), lambda i: (0, 0)),
            pl.BlockSpec((D, D), lambda i: (0, 0)),
            pl.BlockSpec((1, D), lambda i: (0, 0)),
            pl.BlockSpec((1, D), lambda i: (0, 0)),
            pl.BlockSpec((1, D), lambda i: (0, 0)),
            pl.BlockSpec((D, OUT), lambda i: (0, 0)),
            pl.BlockSpec((1, OUT), lambda i: (0, 0)),
        ],
        out_specs=pl.BlockSpec((B, OUT), lambda i: (0, 0)),
        out_shape=jax.ShapeDtypeStruct((B, OUT), jnp.float32),
        scratch_shapes=[
            pltpu.VMEM((B, D), jnp.float32),
            pltpu.VMEM((1, B), jnp.float32),
        ],
    )(x, p, batch.reshape(N // _BN, 1, _BN), W1, row(b1), row(g1), row(be1),
      W2, row(b2), row(g2), row(be2), Wp, bp.reshape(1, OUT))


def kernel(x, edge_index, edge_attr, batch,
           We0, bE0, W10, b10, g10, be10, W20, b20, g20, be20,
           We1, bE1, W11, b11, g11, be11, W21, b21, g21, be21, Wp, bp):
    pad = EPW - E // NW
    src3 = jnp.pad(edge_index[0].reshape(NW, E // NW), ((0, 0), (0, pad)),
                   constant_values=0).reshape(NW, NCHUNK, CH)
    dst3 = jnp.pad(edge_index[1].reshape(NW, E // NW), ((0, 0), (0, pad)),
                   constant_values=DUMP).reshape(NW, NCHUNK, CH)
    eap = jnp.pad(edge_attr.reshape(NW, E // NW, ED),
                  ((0, 0), (0, pad), (0, 0))).reshape(EP, ED)
    ee0, ee1 = _edge_embed(eap, We0, bE0, We1, bE1)
    p0 = _msgpass(x, ee0, src3, dst3)
    x1 = _node_mlp(x, p0, W10, b10, g10, be10, W20, b20, g20, be20)
    p1 = _msgpass(x1, ee1, src3, dst3)
    return _final(x1, p1, batch, W11, b11, g11, be11,
                  W21, b21, g21, be21, Wp, bp)


# speedup vs baseline: 1.3874x; 1.3874x over previous
import functools

import jax
import jax.numpy as jnp
from jax import lax
from jax.experimental import pallas as pl
from jax.experimental.pallas import tpu as pltpu
from jax.experimental.pallas import tpu_sc as plsc

N = 10000
E = 320000
D = 128
ED = 16
H = 128
OUT = 128
B = 256

NC = 2
NS = 16
NW = NC * NS
CH = 128
GRP = 40
NG = 2
NCHUNK = GRP * NG
EPW = NCHUNK * CH
EP = NW * EPW
NP_ = 10112
DUMP = N
RPS = NP_ // NS


def _ln(t, g, b, eps=1e-5):
    m = jnp.mean(t, axis=-1, keepdims=True)
    v = jnp.mean((t - m) ** 2, axis=-1, keepdims=True)
    return (t - m) / jnp.sqrt(v + eps) * g + b


_BE = 2048


def _edge_embed_body(ea_ref, We0_ref, bE0_ref, We1_ref, bE1_ref,
                     ee0_ref, ee1_ref):
    ea = ea_ref[...]
    ee0_ref[...] = jnp.dot(ea, We0_ref[...],
                           preferred_element_type=jnp.float32) + bE0_ref[...]
    ee1_ref[...] = jnp.dot(ea, We1_ref[...],
                           preferred_element_type=jnp.float32) + bE1_ref[...]


def _edge_embed(edge_attr, We0, bE0, We1, bE1):
    grid = (EP // _BE,)
    return pl.pallas_call(
        _edge_embed_body,
        grid=grid,
        in_specs=[
            pl.BlockSpec((_BE, ED), lambda i: (i, 0)),
            pl.BlockSpec((ED, D), lambda i: (0, 0)),
            pl.BlockSpec((1, D), lambda i: (0, 0)),
            pl.BlockSpec((ED, D), lambda i: (0, 0)),
            pl.BlockSpec((1, D), lambda i: (0, 0)),
        ],
        out_specs=[
            pl.BlockSpec((_BE, D), lambda i: (i, 0)),
            pl.BlockSpec((_BE, D), lambda i: (i, 0)),
        ],
        out_shape=[
            jax.ShapeDtypeStruct((EP, D), jnp.float32),
            jax.ShapeDtypeStruct((EP, D), jnp.float32),
        ],
    )(edge_attr, We0, bE0.reshape(1, D), We1, bE1.reshape(1, D))


def _msgpass_body(x_hbm, ee_hbm, src_hbm, dst_hbm, out_hbm,
                  src_g, dst_g, xbuf, eebuf, shared, gsem):
    cid = lax.axis_index("c")
    sid = lax.axis_index("s")
    wid = cid * NS + sid

    def zrow(r, _):
        for j in range(D // 16):
            xbuf[r, pl.ds(j * 16, 16)] = jnp.zeros((16,), jnp.float32)
        return 0
    lax.fori_loop(0, CH, zrow, 0)
    for k in range(RPS // CH):
        pltpu.sync_copy(xbuf, shared.at[pl.ds(sid * RPS + k * CH, CH)])
    rem = RPS % CH
    if rem:
        pltpu.sync_copy(xbuf.at[pl.ds(0, rem)],
                        shared.at[pl.ds(sid * RPS + (RPS // CH) * CH, rem)])

    plsc.subcore_barrier()

    def group(g, _):
        pltpu.sync_copy(src_hbm.at[wid, pl.ds(g * GRP, GRP)], src_g)
        pltpu.sync_copy(dst_hbm.at[wid, pl.ds(g * GRP, GRP)], dst_g)

        def chunk(c, _):
            pltpu.async_copy(x_hbm.at[src_g.at[c]], xbuf, gsem)
            pltpu.sync_copy(
                ee_hbm.at[pl.ds(wid * EPW + (g * GRP + c) * CH, CH)], eebuf)
            pltpu.make_async_copy(x_hbm.at[pl.ds(0, CH)], xbuf, gsem).wait()

            def row(r, _):
                for jj in range(D // 16):
                    sl = pl.ds(jj * 16, 16)
                    eebuf[r, sl] = jnp.maximum(xbuf[r, sl] + eebuf[r, sl], 0.0)
                return 0
            lax.fori_loop(0, CH, row, 0)

            pltpu.sync_copy(eebuf, shared.at[dst_g.at[c]], add=True)
            return 0

        lax.fori_loop(0, GRP, chunk, 0)
        return 0

    lax.fori_loop(0, NG, group, 0)

    plsc.subcore_barrier()

    pltpu.sync_copy(shared.at[pl.ds(sid * RPS, RPS)],
                    out_hbm.at[cid, pl.ds(sid * RPS, RPS)])


def _msgpass(x, ee, src3, dst3):
    mesh = plsc.VectorSubcoreMesh(core_axis_name="c", subcore_axis_name="s",
                                  num_cores=NC, num_subcores=NS)
    f = pl.kernel(
        _msgpass_body,
        out_type=jax.ShapeDtypeStruct((NC, NP_, D), jnp.float32),
        mesh=mesh,
        scratch_types=[
            pltpu.VMEM((GRP, CH), jnp.int32),
            pltpu.VMEM((GRP, CH), jnp.int32),
            pltpu.VMEM((CH, D), jnp.float32),
            pltpu.VMEM((CH, D), jnp.float32),
            pltpu.VMEM_SHARED((NP_, D), jnp.float32),
            pltpu.SemaphoreType.DMA,
        ],
    )
    return f(x, ee, src3, dst3)


_BN = 2000


def _node_mlp_body(x_ref, p_ref, W1_ref, b1_ref, g1_ref, be1_ref,
                   W2_ref, b2_ref, g2_ref, be2_ref, o_ref):
    h = x_ref[...] + p_ref[0] + p_ref[1]
    t = jnp.dot(h, W1_ref[...], preferred_element_type=jnp.float32) + b1_ref[...]
    t = jnp.maximum(_ln(t, g1_ref[...], be1_ref[...]), 0.0)
    u = jnp.dot(t, W2_ref[...], preferred_element_type=jnp.float32) + b2_ref[...]
    o_ref[...] = jnp.maximum(_ln(u, g2_ref[...], be2_ref[...]), 0.0)


def _node_mlp(x, p, W1, b1, g1, be1, W2, b2, g2, be2):
    grid = (N // _BN,)
    row = lambda v: v.reshape(1, D)
    return pl.pallas_call(
        _node_mlp_body,
        grid=grid,
        in_specs=[
            pl.BlockSpec((_BN, D), lambda i: (i, 0)),
            pl.BlockSpec((NC, _BN, D), lambda i: (0, i, 0)),
            pl.BlockSpec((D, D), lambda i: (0, 0)),
            pl.BlockSpec((1, D), lambda i: (0, 0)),
            pl.BlockSpec((1, D), lambda i: (0, 0)),
            pl.BlockSpec((1, D), lambda i: (0, 0)),
            pl.BlockSpec((D, D), lambda i: (0, 0)),
            pl.BlockSpec((1, D), lambda i: (0, 0)),
            pl.BlockSpec((1, D), lambda i: (0, 0)),
            pl.BlockSpec((1, D), lambda i: (0, 0)),
        ],
        out_specs=pl.BlockSpec((_BN, D), lambda i: (i, 0)),
        out_shape=jax.ShapeDtypeStruct((N, D), jnp.float32),
    )(x, p, W1, row(b1), row(g1), row(be1), W2, row(b2), row(g2), row(be2))


def _final_body(x_ref, p_ref, batch_ref, W1_ref, b1_ref, g1_ref, be1_ref,
                W2_ref, b2_ref, g2_ref, be2_ref, Wp_ref, bp_ref,
                o_ref, pooled_acc, counts_acc):
    i = pl.program_id(0)
    h = x_ref[...] + p_ref[0] + p_ref[1]
    t = jnp.dot(h, W1_ref[...], preferred_element_type=jnp.float32) + b1_ref[...]
    t = jnp.maximum(_ln(t, g1_ref[...], be1_ref[...]), 0.0)
    u = jnp.dot(t, W2_ref[...], preferred_element_type=jnp.float32) + b2_ref[...]
    x2 = jnp.maximum(_ln(u, g2_ref[...], be2_ref[...]), 0.0)

    bvec = batch_ref[0, 0, :]
    mask = (bvec[:, None] ==
            lax.broadcasted_iota(jnp.int32, (_BN, B), 1)).astype(jnp.float32)
    pm = lax.dot_general(mask, x2, (((0,), (0,)), ((), ())),
                         preferred_element_type=jnp.float32)
    cm = jnp.sum(mask, axis=0)[None, :]

    @pl.when(i == 0)
    def _():
        pooled_acc[...] = jnp.zeros_like(pooled_acc)
        counts_acc[...] = jnp.zeros_like(counts_acc)

    pooled_acc[...] += pm
    counts_acc[...] += cm

    @pl.when(i == pl.num_programs(0) - 1)
    def _():
        cnt = jnp.maximum(counts_acc[0, :], 1.0)
        proj = jnp.dot(pooled_acc[...], Wp_ref[...],
                       preferred_element_type=jnp.float32)
        o_ref[...] = proj / cnt[:, None] + bp_ref[...]


def _final(x, p, batch, W1, b1, g1, be1, W2, b2, g2, be2, Wp, bp):
    grid = (N // _BN,)
    row = lambda v: v.reshape(1, D)
    return pl.pallas_call(
        _final_body,
        grid=grid,
        in_specs=[
            pl.BlockSpec((_BN, D), lambda i: (i, 0)),
            pl.BlockSpec((NC, _BN, D), lambda i: (0, i, 0)),
            pl.BlockSpec((1, 1, _BN), lambda i: (i, 0, 0)),
            pl.BlockSpec((D, D), lambda i: (0, 0)),
            pl.BlockSpec((1, D), lambda i: (0, 0)),
            pl.BlockSpec((1, D), lambda i: (0, 0)),
            pl.BlockSpec((1, D), lambda i: (0, 0)),
            pl.BlockSpec((D, D), lambda i: (0, 0)),
            pl.BlockSpec((1, D), lambda i: (0, 0)),
            pl.BlockSpec((1, D), lambda i: (0, 0)),
            pl.BlockSpec((1, D), lambda i: (0, 0)),
            pl.BlockSpec((D, OUT), lambda i: (0, 0)),
            pl.BlockSpec((1, OUT), lambda i: (0, 0)),
        ],
        out_specs=pl.BlockSpec((B, OUT), lambda i: (0, 0)),
        out_shape=jax.ShapeDtypeStruct((B, OUT), jnp.float32),
        scratch_shapes=[
            pltpu.VMEM((B, D), jnp.float32),
            pltpu.VMEM((1, B), jnp.float32),
        ],
    )(x, p, batch.reshape(N // _BN, 1, _BN), W1, row(b1), row(g1), row(be1),
      W2, row(b2), row(g2), row(be2), Wp, bp.reshape(1, OUT))


def kernel(x, edge_index, edge_attr, batch,
           We0, bE0, W10, b10, g10, be10, W20, b20, g20, be20,
           We1, bE1, W11, b11, g11, be11, W21, b21, g21, be21, Wp, bp):
    pad = EPW - E // NW
    src3 = jnp.pad(edge_index[0].reshape(NW, E // NW), ((0, 0), (0, pad)),
                   constant_values=0).reshape(NW, NCHUNK, CH)
    dst3 = jnp.pad(edge_index[1].reshape(NW, E // NW), ((0, 0), (0, pad)),
                   constant_values=DUMP).reshape(NW, NCHUNK, CH)
    eap = jnp.pad(edge_attr.reshape(NW, E // NW, ED),
                  ((0, 0), (0, pad), (0, 0))).reshape(EP, ED)
    ee0, ee1 = _edge_embed(eap, We0, bE0, We1, bE1)
    p0 = _msgpass(x, ee0, src3, dst3)
    x1 = _node_mlp(x, p0, W10, b10, g10, be10, W20, b20, g20, be20)
    p1 = _msgpass(x1, ee1, src3, dst3)
    return _final(x1, p1, batch, W11, b11, g11, be11,
                  W21, b21, g21, be21, Wp, bp)
